# baseline (speedup 1.0000x reference)
import functools

import jax
import jax.numpy as jnp
from jax import lax
from jax.experimental import pallas as pl
from jax.experimental.pallas import tpu as pltpu
from jax.experimental.pallas import tpu_sc as plsc

_NC = 2
_NS = 16
_NW = _NC * _NS
_LANES = 16


def _sc_slab_gather(table, dxbc, c, h_total):
    cp = table.shape[1]
    per_w = h_total // _NW
    rows_per_w = per_w * c
    k = 112
    n_chunks = rows_per_w // k
    groups_per_chunk = k // _LANES

    mesh = plsc.VectorSubcoreMesh(
        core_axis_name="c", subcore_axis_name="s",
        num_cores=_NC, num_subcores=_NS)

    @functools.partial(
        pl.kernel,
        mesh=mesh,
        out_type=jax.ShapeDtypeStruct((h_total * c, cp), jnp.float32),
        scratch_types=[
            pltpu.VMEM((8, _LANES), jnp.int32),
            pltpu.VMEM((k,), jnp.int32),
            pltpu.VMEM((k,), jnp.int32),
            pltpu.VMEM((k, cp), jnp.float32),
            pltpu.VMEM((k, cp), jnp.float32),
            pltpu.SemaphoreType.DMA,
            pltpu.SemaphoreType.DMA,
            pltpu.SemaphoreType.DMA,
            pltpu.SemaphoreType.DMA,
        ],
    )
    def gather_kernel(table_hbm, dxbc_hbm, out_hbm,
                      dxv, idx_a, idx_b, rows_a, rows_b,
                      isem_a, isem_b, osem_a, osem_b):
        wid = lax.axis_index("s") * _NC + lax.axis_index("c")
        h0 = wid * per_w
        tile_base = h0 * c
        pltpu.sync_copy(dxbc_hbm.at[wid], dxv)

        iota16 = lax.broadcasted_iota(jnp.int32, (_LANES,), 0)
        idx_bufs = (idx_a, idx_b)
        row_bufs = (rows_a, rows_b)
        isems = (isem_a, isem_b)
        osems = (osem_a, osem_b)

        def build_idx(m):
            buf = idx_bufs[m % 2]
            for g in range(groups_per_chunk):
                r0 = m * k + g * _LANES
                dxb = dxv[r0 // c]
                buf[pl.ds(g * _LANES, _LANES)] = dxb * c + (r0 % c) + iota16

        def start_in(m):
            return pltpu.async_copy(
                table_hbm.at[idx_bufs[m % 2]], row_bufs[m % 2], isems[m % 2])

        def start_out(m):
            return pltpu.async_copy(
                row_bufs[m % 2],
                out_hbm.at[pl.ds(tile_base + m * k, k)], osems[m % 2])

        build_idx(0)
        d_in = {0: start_in(0)}
        d_out = {}
        for m in range(n_chunks):
            if m + 1 < n_chunks:
                build_idx(m + 1)
            d_in[m].wait()
            if m + 1 < n_chunks:
                if m - 1 >= 0:
                    d_out[m - 1].wait()
                d_in[m + 1] = start_in(m + 1)
            d_out[m] = start_out(m)
        d_out[n_chunks - 2].wait()
        d_out[n_chunks - 1].wait()

    return gather_kernel(table, dxbc)


def _tc_weighted_mul(dy0col, xt, sw3):
    b, h, c, w = xt.shape
    cp = sw3.shape[-1]

    hb = 8

    def mul_kernel(dy_ref, x_ref, w_ref, o_ref):
        i = pl.program_id(0)
        for j in range(hb):
            dy0 = dy_ref[i * hb + j]
            rolled = pltpu.roll(w_ref[j], -dy0, 1)[:, :w]
            o_ref[:, j] = x_ref[:, j] * jax.nn.sigmoid(rolled)[None]

    grid_spec = pltpu.PrefetchScalarGridSpec(
        num_scalar_prefetch=1,
        grid=(h // hb,),
        in_specs=[
            pl.BlockSpec((b, hb, c, w), lambda i, d: (0, i, 0, 0)),
            pl.BlockSpec((hb, c, cp), lambda i, d: (i, 0, 0)),
        ],
        out_specs=pl.BlockSpec((b, hb, c, w), lambda i, d: (0, i, 0, 0)),
    )
    return pl.pallas_call(
        mul_kernel,
        grid_spec=grid_spec,
        out_shape=jax.ShapeDtypeStruct((b, h, c, w), jnp.float32),
    )(dy0col, xt, sw3)


def kernel(x, pos_weights, dx_indices, dy_indices):
    b, h, w, c = x.shape
    th, tw, _ = pos_weights.shape

    xt = jnp.swapaxes(x, 2, 3)
    pwt2 = jnp.swapaxes(pos_weights, 1, 2).reshape(th * c, tw)
    table = jnp.pad(pwt2, ((0, 0), (0, 256 - tw)))
    dxcol = jnp.pad(dx_indices[:, 0], (0, 8))
    rowsel = jnp.arange(_NW)[:, None] * (h // _NW) + jnp.arange(8)[None, :]
    dxbc = jnp.broadcast_to(dxcol[rowsel][..., None], (_NW, 8, _LANES))
    dy0col = dy_indices[:, 0]

    sw = _sc_slab_gather(table, dxbc, c, h)
    sw3 = sw.reshape(h, c, 256)

    out_t = _tc_weighted_mul(dy0col, xt, sw3)
    return jnp.swapaxes(out_t, 2, 3)


# speedup vs baseline: 1.0023x; 1.0023x over previous
import functools

import jax
import jax.numpy as jnp
from jax import lax
from jax.experimental import pallas as pl
from jax.experimental.pallas import tpu as pltpu
from jax.experimental.pallas import tpu_sc as plsc

_NC = 2
_NS = 16
_NW = _NC * _NS
_LANES = 16


def _sc_slab_gather(table, dxbc, c, h_count):
    cp = table.shape[1]
    per_w = h_count // _NW
    rows_per_w = per_w * c
    k = 96
    n_chunks = rows_per_w // k
    groups_per_chunk = k // _LANES
    assert rows_per_w % k == 0 and per_w <= 8

    mesh = plsc.VectorSubcoreMesh(
        core_axis_name="c", subcore_axis_name="s",
        num_cores=_NC, num_subcores=_NS)

    @functools.partial(
        pl.kernel,
        mesh=mesh,
        out_type=jax.ShapeDtypeStruct((h_count * c, cp), jnp.float32),
        scratch_types=[
            pltpu.VMEM((8, _LANES), jnp.int32),
            pltpu.VMEM((k,), jnp.int32),
            pltpu.VMEM((k,), jnp.int32),
            pltpu.VMEM((k, cp), jnp.float32),
            pltpu.VMEM((k, cp), jnp.float32),
            pltpu.SemaphoreType.DMA,
            pltpu.SemaphoreType.DMA,
            pltpu.SemaphoreType.DMA,
            pltpu.SemaphoreType.DMA,
        ],
    )
    def gather_kernel(table_hbm, dxbc_hbm, out_hbm,
                      dxv, idx_a, idx_b, rows_a, rows_b,
                      isem_a, isem_b, osem_a, osem_b):
        wid = lax.axis_index("s") * _NC + lax.axis_index("c")
        tile_base = wid * rows_per_w
        pltpu.sync_copy(dxbc_hbm.at[wid], dxv)

        iota16 = lax.broadcasted_iota(jnp.int32, (_LANES,), 0)
        idx_bufs = (idx_a, idx_b)
        row_bufs = (rows_a, rows_b)
        isems = (isem_a, isem_b)
        osems = (osem_a, osem_b)

        def build_idx(m):
            buf = idx_bufs[m % 2]
            for g in range(groups_per_chunk):
                r0 = m * k + g * _LANES
                dxb = dxv[r0 // c]
                buf[pl.ds(g * _LANES, _LANES)] = dxb * c + (r0 % c) + iota16

        def start_in(m):
            return pltpu.async_copy(
                table_hbm.at[idx_bufs[m % 2]], row_bufs[m % 2], isems[m % 2])

        def start_out(m):
            return pltpu.async_copy(
                row_bufs[m % 2],
                out_hbm.at[pl.ds(tile_base + m * k, k)], osems[m % 2])

        build_idx(0)
        d_in = {0: start_in(0)}
        d_out = {}
        for m in range(n_chunks):
            if m + 1 < n_chunks:
                build_idx(m + 1)
            d_in[m].wait()
            if m + 1 < n_chunks:
                if m - 1 >= 0:
                    d_out[m - 1].wait()
                d_in[m + 1] = start_in(m + 1)
            d_out[m] = start_out(m)
        d_out[n_chunks - 2].wait()
        d_out[n_chunks - 1].wait()

    return gather_kernel(table, dxbc)


def _make_dxbc(dxcol, h_start, per_w):
    rowsel = h_start + jnp.arange(_NW)[:, None] * per_w + jnp.arange(8)[None, :]
    return jnp.broadcast_to(
        dxcol[jnp.minimum(rowsel, dxcol.shape[0] - 1)][..., None],
        (_NW, 8, _LANES))


def _tc_weighted_mul(dy0col, xt, sw3, h_start, h_count, prev=None):
    b, h, c, w = xt.shape
    cp = sw3.shape[-1]
    hb = 8
    blk0 = h_start // hb

    def mul_kernel(dy_ref, x_ref, w_ref, *rest):
        o_ref = rest[-1]
        i = pl.program_id(0)
        for j in range(hb):
            dy0 = dy_ref[(blk0 + i) * hb + j]
            rolled = pltpu.roll(w_ref[j], -dy0, 1)[:, :w]
            o_ref[:, j] = x_ref[:, j] * jax.nn.sigmoid(rolled)[None]

    in_specs = [
        pl.BlockSpec((b, hb, c, w), lambda i, d: (0, blk0 + i, 0, 0)),
        pl.BlockSpec((hb, c, cp), lambda i, d: (i, 0, 0)),
    ]
    operands = [xt, sw3]
    kwargs = {}
    if prev is not None:
        in_specs.append(pl.BlockSpec(memory_space=pl.ANY))
        operands.append(prev)
        kwargs["input_output_aliases"] = {3: 0}

    grid_spec = pltpu.PrefetchScalarGridSpec(
        num_scalar_prefetch=1,
        grid=(h_count // hb,),
        in_specs=in_specs,
        out_specs=pl.BlockSpec((b, hb, c, w), lambda i, d: (0, blk0 + i, 0, 0)),
    )
    return pl.pallas_call(
        mul_kernel,
        grid_spec=grid_spec,
        out_shape=jax.ShapeDtypeStruct((b, h, c, w), jnp.float32),
        **kwargs,
    )(dy0col, *operands)


def kernel(x, pos_weights, dx_indices, dy_indices):
    b, h, w, c = x.shape
    th, tw, _ = pos_weights.shape
    h_a = 64
    h_b = h - h_a

    xt = jnp.swapaxes(x, 2, 3)
    pwt2 = jnp.swapaxes(pos_weights, 1, 2).reshape(th * c, tw)
    table = jnp.pad(pwt2, ((0, 0), (0, 256 - tw)))
    dxcol = jnp.pad(dx_indices[:, 0], (0, 8))
    dy0col = dy_indices[:, 0]

    sw_a = _sc_slab_gather(table, _make_dxbc(dxcol, 0, h_a // _NW), c, h_a)
    sw_b = _sc_slab_gather(table, _make_dxbc(dxcol, h_a, h_b // _NW), c, h_b)

    out_a = _tc_weighted_mul(dy0col, xt, sw_a.reshape(h_a, c, 256), 0, h_a)
    out_t = _tc_weighted_mul(dy0col, xt, sw_b.reshape(h_b, c, 256),
                             h_a, h_b, prev=out_a)
    return jnp.swapaxes(out_t, 2, 3)


# speedup vs baseline: 1.0030x; 1.0007x over previous
import functools

import jax
import jax.numpy as jnp
from jax import lax
from jax.experimental import pallas as pl
from jax.experimental.pallas import tpu as pltpu
from jax.experimental.pallas import tpu_sc as plsc

_NC = 2
_NS = 16
_NW = _NC * _NS
_LANES = 16


def _sc_slab_gather(table, dxbc, c, h_total):
    cp = table.shape[1]
    per_w = h_total // _NW
    rows_per_w = per_w * c
    k = 112
    n_chunks = rows_per_w // k
    groups_per_chunk = k // _LANES

    mesh = plsc.VectorSubcoreMesh(
        core_axis_name="c", subcore_axis_name="s",
        num_cores=_NC, num_subcores=_NS)

    @functools.partial(
        pl.kernel,
        mesh=mesh,
        out_type=jax.ShapeDtypeStruct((h_total * c, cp), jnp.float32),
        scratch_types=[
            pltpu.VMEM((8, _LANES), jnp.int32),
            pltpu.VMEM((k,), jnp.int32),
            pltpu.VMEM((k,), jnp.int32),
            pltpu.VMEM((k, cp), jnp.float32),
            pltpu.VMEM((k, cp), jnp.float32),
            pltpu.SemaphoreType.DMA,
            pltpu.SemaphoreType.DMA,
            pltpu.SemaphoreType.DMA,
            pltpu.SemaphoreType.DMA,
        ],
    )
    def gather_kernel(table_hbm, dxbc_hbm, out_hbm,
                      dxv, idx_a, idx_b, rows_a, rows_b,
                      isem_a, isem_b, osem_a, osem_b):
        wid = lax.axis_index("s") * _NC + lax.axis_index("c")
        h0 = wid * per_w
        tile_base = h0 * c
        pltpu.sync_copy(dxbc_hbm.at[wid], dxv)

        iota16 = lax.broadcasted_iota(jnp.int32, (_LANES,), 0)
        idx_bufs = (idx_a, idx_b)
        row_bufs = (rows_a, rows_b)
        isems = (isem_a, isem_b)
        osems = (osem_a, osem_b)

        def build_idx(m):
            buf = idx_bufs[m % 2]
            for g in range(groups_per_chunk):
                r0 = m * k + g * _LANES
                dxb = dxv[r0 // c]
                buf[pl.ds(g * _LANES, _LANES)] = dxb * c + (r0 % c) + iota16

        def start_in(m):
            return pltpu.async_copy(
                table_hbm.at[idx_bufs[m % 2]], row_bufs[m % 2], isems[m % 2])

        def start_out(m):
            return pltpu.async_copy(
                row_bufs[m % 2],
                out_hbm.at[pl.ds(tile_base + m * k, k)], osems[m % 2])

        build_idx(0)
        d_in = {0: start_in(0)}
        d_out = {}
        for m in range(n_chunks):
            if m + 1 < n_chunks:
                build_idx(m + 1)
            d_in[m].wait()
            if m + 1 < n_chunks:
                if m - 1 >= 0:
                    d_out[m - 1].wait()
                d_in[m + 1] = start_in(m + 1)
            d_out[m] = start_out(m)
        d_out[n_chunks - 2].wait()
        d_out[n_chunks - 1].wait()

    return gather_kernel(table, dxbc)


def _tc_weighted_mul(dy0col, xt, sw3):
    b, h, c, w = xt.shape
    cp = sw3.shape[-1]

    hb = 7

    def mul_kernel(dy_ref, x_ref, w_ref, o_ref):
        i = pl.program_id(0)
        for j in range(hb):
            dy0 = dy_ref[i * hb + j]
            rolled = pltpu.roll(w_ref[j], -dy0, 1)[:, :w]
            o_ref[:, j] = x_ref[:, j] * jax.nn.sigmoid(rolled)[None]

    grid_spec = pltpu.PrefetchScalarGridSpec(
        num_scalar_prefetch=1,
        grid=(h // hb,),
        in_specs=[
            pl.BlockSpec((b, hb, c, w), lambda i, d: (0, i, 0, 0)),
            pl.BlockSpec((hb, c, cp), lambda i, d: (i, 0, 0)),
        ],
        out_specs=pl.BlockSpec((b, hb, c, w), lambda i, d: (0, i, 0, 0)),
    )
    return pl.pallas_call(
        mul_kernel,
        grid_spec=grid_spec,
        out_shape=jax.ShapeDtypeStruct((b, h, c, w), jnp.float32),
    )(dy0col, xt, sw3)


def kernel(x, pos_weights, dx_indices, dy_indices):
    b, h, w, c = x.shape
    th, tw, _ = pos_weights.shape

    xt = jnp.swapaxes(x, 2, 3)
    pwt2 = jnp.swapaxes(pos_weights, 1, 2).reshape(th * c, tw)
    table = jnp.pad(pwt2, ((0, 0), (0, 256 - tw)))
    dxcol = jnp.pad(dx_indices[:, 0], (0, 8))
    rowsel = jnp.arange(_NW)[:, None] * (h // _NW) + jnp.arange(8)[None, :]
    dxbc = jnp.broadcast_to(dxcol[rowsel][..., None], (_NW, 8, _LANES))
    dy0col = dy_indices[:, 0]

    sw = _sc_slab_gather(table, dxbc, c, h)
    sw3 = sw.reshape(h, c, 256)

    out_t = _tc_weighted_mul(dy0col, xt, sw3)
    return jnp.swapaxes(out_t, 2, 3)
